# Initial kernel scaffold; baseline (speedup 1.0000x reference)
import functools

import jax
import jax.numpy as jnp
from jax import lax
from jax.experimental import pallas as pl
from jax.experimental.pallas import tpu as pltpu
from jax.experimental.pallas import tpu_sc as plsc

N_FIELDS = 26
VOCAB = 100000
EMB_DIM = 32
BATCH = 16384

NC = 2
NS = 16
NW = NC * NS
TOTAL_ROWS = BATCH * N_FIELDS
PER_W = TOTAL_ROWS // NW
NCHUNK = 8
CHUNK = PER_W // NCHUNK
VECS = PER_W // 16


def _body(tab_hbm, idx_hbm, out_hbm, idx_v, buf0, buf1, gs0, gs1, os0, os1):
    wid = lax.axis_index("s") * NC + lax.axis_index("c")
    base = wid * PER_W

    pltpu.sync_copy(idx_hbm.at[pl.ds(base, PER_W)], idx_v)

    iota = jnp.arange(16, dtype=jnp.int32)
    def fix(j, carry):
        v = idx_v[pl.ds(j * 16, 16)]
        f = lax.rem(j * 16 + iota, N_FIELDS)
        idx_v[pl.ds(j * 16, 16)] = v + f * VOCAB
        return carry
    lax.fori_loop(0, VECS, fix, 0)

    bufs = (buf0, buf1)
    gsems = (gs0, gs1)
    osems = (os0, os1)

    def gather(c):
        return pltpu.async_copy(
            tab_hbm.at[idx_v.at[pl.ds(c * CHUNK, CHUNK)]], bufs[c % 2],
            gsems[c % 2])

    def copy_out(c):
        return pltpu.async_copy(
            bufs[c % 2], out_hbm.at[pl.ds(base + c * CHUNK, CHUNK)],
            osems[c % 2])

    g = {0: gather(0)}
    o = {}
    for c in range(NCHUNK):
        if c + 1 < NCHUNK:
            if c - 1 >= 0:
                o[c - 1].wait()
            g[c + 1] = gather(c + 1)
        g[c].wait()
        o[c] = copy_out(c)
    o[NCHUNK - 2].wait()
    o[NCHUNK - 1].wait()


_sc_gather = pl.kernel(
    _body,
    out_type=jax.ShapeDtypeStruct((TOTAL_ROWS, EMB_DIM), jnp.float32),
    mesh=plsc.VectorSubcoreMesh(
        core_axis_name="c", subcore_axis_name="s",
        num_cores=NC, num_subcores=NS),
    scratch_types=[
        pltpu.VMEM((PER_W,), jnp.int32),
        pltpu.VMEM((CHUNK, EMB_DIM), jnp.float32),
        pltpu.VMEM((CHUNK, EMB_DIM), jnp.float32),
        pltpu.SemaphoreType.DMA,
        pltpu.SemaphoreType.DMA,
        pltpu.SemaphoreType.DMA,
        pltpu.SemaphoreType.DMA,
    ],
)


@jax.jit
def kernel(categorical_data, tables):
    cat = categorical_data.astype(jnp.int32).reshape(TOTAL_ROWS)
    tab = tables.reshape(N_FIELDS * VOCAB, EMB_DIM)
    out = _sc_gather(tab, cat)
    return out.reshape(BATCH, N_FIELDS * EMB_DIM)


# speedup vs baseline: 1.2143x; 1.2143x over previous
import functools

import jax
import jax.numpy as jnp
from jax import lax
from jax.experimental import pallas as pl
from jax.experimental.pallas import tpu as pltpu
from jax.experimental.pallas import tpu_sc as plsc

N_FIELDS = 26
VOCAB = 100000
EMB_DIM = 32
BATCH = 16384

NC = 2
NS = 16
NW = NC * NS
TOTAL_ROWS = BATCH * N_FIELDS
PER_W = TOTAL_ROWS // NW
NCHUNK = 8
CHUNK = PER_W // NCHUNK
VECS = PER_W // 16


def _body(tab_hbm, idx_hbm, out_hbm, idx_v, buf0, buf1, gs0, gs1, os0, os1):
    wid = lax.axis_index("s") * NC + lax.axis_index("c")
    base = wid * PER_W

    pltpu.sync_copy(idx_hbm.at[pl.ds(base, PER_W)], idx_v)

    iota = jnp.arange(16, dtype=jnp.int32)
    def fix(j, carry):
        v = idx_v[pl.ds(j * 16, 16)]
        f = lax.rem(j * 16 + iota, N_FIELDS)
        idx_v[pl.ds(j * 16, 16)] = v + f * VOCAB
        return carry
    lax.fori_loop(0, VECS, fix, 0)

    bufs = (buf0, buf1)
    gsems = (gs0, gs1)
    osems = (os0, os1)

    def gather(c):
        return pltpu.async_copy(
            tab_hbm.at[idx_v.at[pl.ds(c * CHUNK, CHUNK)]], bufs[c % 2],
            gsems[c % 2])

    def copy_out(c):
        return pltpu.async_copy(
            bufs[c % 2], out_hbm.at[pl.ds(base + c * CHUNK, CHUNK)],
            osems[c % 2])

    g = {0: gather(0)}
    o = {}
    for c in range(NCHUNK):
        if c + 1 < NCHUNK:
            if c - 1 >= 0:
                o[c - 1].wait()
            g[c + 1] = gather(c + 1)
        g[c].wait()
        o[c] = copy_out(c)
    o[NCHUNK - 2].wait()
    o[NCHUNK - 1].wait()


_sc_gather = pl.kernel(
    _body,
    out_type=jax.ShapeDtypeStruct((TOTAL_ROWS, EMB_DIM), jnp.float32),
    mesh=plsc.VectorSubcoreMesh(
        core_axis_name="c", subcore_axis_name="s",
        num_cores=NC, num_subcores=NS),
    scratch_types=[
        pltpu.VMEM((PER_W,), jnp.int32),
        pltpu.VMEM((CHUNK, EMB_DIM), jnp.float32),
        pltpu.VMEM((CHUNK, EMB_DIM), jnp.float32),
        pltpu.SemaphoreType.DMA,
        pltpu.SemaphoreType.DMA,
        pltpu.SemaphoreType.DMA,
        pltpu.SemaphoreType.DMA,
    ],
    compiler_params=pltpu.CompilerParams(use_tc_tiling_on_sc=False),
)


@jax.jit
def kernel(categorical_data, tables):
    cat = categorical_data.astype(jnp.int32).reshape(TOTAL_ROWS)
    tab = tables.reshape(N_FIELDS * VOCAB, EMB_DIM)
    out = _sc_gather(tab, cat)
    return out.reshape(BATCH, N_FIELDS * EMB_DIM)


# speedup vs baseline: 3.9936x; 3.2888x over previous
import functools

import jax
import jax.numpy as jnp
from jax import lax
from jax.experimental import pallas as pl
from jax.experimental.pallas import tpu as pltpu
from jax.experimental.pallas import tpu_sc as plsc

N_FIELDS = 26
VOCAB = 100000
EMB_DIM = 32
BATCH = 16384

NC = 2
NS = 16
NW = NC * NS
N_PAIRS = N_FIELDS * EMB_DIM
PER_W = N_PAIRS // NW
OUT_CHUNK = 4096
N_OCH = BATCH // OUT_CHUNK


def _body(tab_hbm, idx_hbm, out_hbm, vocab_v, idx_v, ob0):
    wid = lax.axis_index("s") * NC + lax.axis_index("c")

    def do_pair(p, carry):
        j = wid * PER_W + p
        i = j // EMB_DIM
        d = j % EMB_DIM
        @pl.when(jnp.logical_or(p == 0, d == 0))
        def _():
            pltpu.sync_copy(idx_hbm.at[i], idx_v)
        pltpu.sync_copy(tab_hbm.at[i, d], vocab_v)

        def do_chunk(c, carry2):
            b0 = c * OUT_CHUNK

            def gather_vecs(k, carry3):
                off = k * 16
                iv = idx_v[pl.ds(b0 + off, 16)]
                ob0[pl.ds(off, 16)] = plsc.load_gather(vocab_v, [iv])
                return carry3

            lax.fori_loop(0, OUT_CHUNK // 16, gather_vecs, 0, unroll=8)
            pltpu.sync_copy(ob0, out_hbm.at[j, pl.ds(b0, OUT_CHUNK)])
            return carry2

        lax.fori_loop(0, N_OCH, do_chunk, 0)
        return carry

    lax.fori_loop(0, PER_W, do_pair, 0)


_sc_col_gather = pl.kernel(
    _body,
    out_type=jax.ShapeDtypeStruct((N_PAIRS, BATCH), jnp.float32),
    mesh=plsc.VectorSubcoreMesh(
        core_axis_name="c", subcore_axis_name="s",
        num_cores=NC, num_subcores=NS),
    scratch_types=[
        pltpu.VMEM((VOCAB,), jnp.float32),
        pltpu.VMEM((BATCH,), jnp.int32),
        pltpu.VMEM((OUT_CHUNK,), jnp.float32),
    ],
    compiler_params=pltpu.CompilerParams(
        use_tc_tiling_on_sc=True, needs_layout_passes=False),
)


@jax.jit
def kernel(categorical_data, tables):
    cat_t = categorical_data.astype(jnp.int32).T
    tab_t = lax.transpose(tables, (0, 2, 1))
    out_t = _sc_col_gather(tab_t, cat_t)
    return out_t.T


# speedup vs baseline: 7.8631x; 1.9689x over previous
import functools

import jax
import jax.numpy as jnp
from jax import lax
from jax.experimental import pallas as pl
from jax.experimental.pallas import tpu as pltpu
from jax.experimental.pallas import tpu_sc as plsc

N_FIELDS = 26
VOCAB = 100000
EMB_DIM = 32
BATCH = 16384

NC = 2
NS = 16
NW = NC * NS
N_PAIRS = N_FIELDS * EMB_DIM
PER_W = N_PAIRS // NW
OUT_CHUNK = 4096
N_OCH = BATCH // OUT_CHUNK


def _body(tab_hbm, idx_hbm, out_hbm, vocab_v, idx_v, ob0):
    wid = lax.axis_index("s") * NC + lax.axis_index("c")

    def do_pair(p, carry):
        j = wid * PER_W + p
        i = j // EMB_DIM
        d = j % EMB_DIM
        @pl.when(jnp.logical_or(p == 0, d == 0))
        def _():
            pltpu.sync_copy(idx_hbm.at[i], idx_v)
        pltpu.sync_copy(tab_hbm.at[i, d], vocab_v)

        def do_chunk(c, carry2):
            b0 = c * OUT_CHUNK

            @plsc.parallel_loop(0, OUT_CHUNK, step=16, unroll=8)
            def _gather(off):
                iv = idx_v[pl.ds(b0 + off, 16)]
                ob0[pl.ds(off, 16)] = plsc.load_gather(vocab_v, [iv])
            pltpu.sync_copy(ob0, out_hbm.at[j, pl.ds(b0, OUT_CHUNK)])
            return carry2

        lax.fori_loop(0, N_OCH, do_chunk, 0)
        return carry

    lax.fori_loop(0, PER_W, do_pair, 0)


_sc_col_gather = pl.kernel(
    _body,
    out_type=jax.ShapeDtypeStruct((N_PAIRS, BATCH), jnp.float32),
    mesh=plsc.VectorSubcoreMesh(
        core_axis_name="c", subcore_axis_name="s",
        num_cores=NC, num_subcores=NS),
    scratch_types=[
        pltpu.VMEM((VOCAB,), jnp.float32),
        pltpu.VMEM((BATCH,), jnp.int32),
        pltpu.VMEM((OUT_CHUNK,), jnp.float32),
    ],
    compiler_params=pltpu.CompilerParams(
        use_tc_tiling_on_sc=True, needs_layout_passes=False),
)


@jax.jit
def kernel(categorical_data, tables):
    cat_t = categorical_data.astype(jnp.int32).T
    tab_t = lax.transpose(tables, (0, 2, 1))
    out_t = _sc_col_gather(tab_t, cat_t)
    return out_t.T


# speedup vs baseline: 8.5470x; 1.0870x over previous
import functools

import jax
import jax.numpy as jnp
from jax import lax
from jax.experimental import pallas as pl
from jax.experimental.pallas import tpu as pltpu
from jax.experimental.pallas import tpu_sc as plsc

N_FIELDS = 26
VOCAB = 100000
EMB_DIM = 32
BATCH = 16384

NC = 2
NS = 16
NW = NC * NS
N_PAIRS = N_FIELDS * EMB_DIM
PER_W = N_PAIRS // NW
OUT_CHUNK = 4096
N_OCH = BATCH // OUT_CHUNK
V_SPLITS = (0, 25088, 50176, 75264, VOCAB)


def _body2(tab_hbm, idx_hbm, out_hbm, vocab_v, idx_v, ob0, ob1, vsem, os0, os1):
    wid = lax.axis_index("s") * NC + lax.axis_index("c")
    obufs = (ob0, ob1)
    osems = (os0, os1)

    def do_pair(p, carry):
        j = wid * PER_W + p
        i = j // EMB_DIM
        d = j % EMB_DIM
        vcp = pltpu.async_copy(tab_hbm.at[i, d], vocab_v, vsem)
        @pl.when(jnp.logical_or(p == 0, d == 0))
        def _():
            pltpu.sync_copy(idx_hbm.at[i], idx_v)
        vcp.wait()

        def do_chunk(c, carry2):
            b0 = c * OUT_CHUNK
            gc = p * N_OCH + c
            buf_sel = c % 2

            def run(buf, sem):
                @pl.when(gc >= 2)
                def _():
                    pltpu.make_async_copy(
                        out_hbm.at[j, pl.ds(b0, OUT_CHUNK)], buf, sem).wait()

                @plsc.parallel_loop(0, OUT_CHUNK, step=16, unroll=8)
                def _gather(off):
                    iv = idx_v[pl.ds(b0 + off, 16)]
                    buf[pl.ds(off, 16)] = plsc.load_gather(vocab_v, [iv])

                pltpu.async_copy(buf, out_hbm.at[j, pl.ds(b0, OUT_CHUNK)], sem)

            @pl.when(buf_sel == 0)
            def _():
                run(ob0, os0)

            @pl.when(buf_sel == 1)
            def _():
                run(ob1, os1)

            return carry2

        lax.fori_loop(0, N_OCH, do_chunk, 0)
        return carry

    lax.fori_loop(0, PER_W, do_pair, 0)
    last_j = wid * PER_W + PER_W - 1
    pltpu.make_async_copy(
        out_hbm.at[last_j, pl.ds(0, OUT_CHUNK)], ob0, os0).wait()
    pltpu.make_async_copy(
        out_hbm.at[last_j, pl.ds(0, OUT_CHUNK)], ob1, os1).wait()


_sc_col_gather = pl.kernel(
    _body2,
    out_type=jax.ShapeDtypeStruct((N_PAIRS, BATCH), jnp.float32),
    mesh=plsc.VectorSubcoreMesh(
        core_axis_name="c", subcore_axis_name="s",
        num_cores=NC, num_subcores=NS),
    scratch_types=[
        pltpu.VMEM((VOCAB,), jnp.float32),
        pltpu.VMEM((BATCH,), jnp.int32),
        pltpu.VMEM((OUT_CHUNK,), jnp.float32),
        pltpu.VMEM((OUT_CHUNK,), jnp.float32),
        pltpu.SemaphoreType.DMA,
        pltpu.SemaphoreType.DMA,
        pltpu.SemaphoreType.DMA,
    ],
    compiler_params=pltpu.CompilerParams(
        use_tc_tiling_on_sc=True, needs_layout_passes=False),
)


@jax.jit
def kernel(categorical_data, tables):
    cat_t = categorical_data.astype(jnp.int32).T
    tab_t = lax.transpose(tables, (0, 2, 1))
    out_t = _sc_col_gather(tab_t, cat_t)
    return out_t.T


# speedup vs baseline: 8.5564x; 1.0011x over previous
import jax
import jax.numpy as jnp
from jax import lax
from jax.experimental import pallas as pl
from jax.experimental.pallas import tpu as pltpu
from jax.experimental.pallas import tpu_sc as plsc

N_FIELDS = 26
VOCAB = 100000
EMB_DIM = 32
BATCH = 16384

NC = 2
NS = 16
NW = NC * NS
N_PAIRS = N_FIELDS * EMB_DIM
PER_W = N_PAIRS // NW
OUT_CHUNK = 4096
N_OCH = BATCH // OUT_CHUNK


def _body(tab_hbm, idx_hbm, out_hbm, vocab_v, idx_v, ob0, ob1, vsem, os0, os1):
    wid = lax.axis_index("s") * NC + lax.axis_index("c")

    def do_pair(p, carry):
        j = wid * PER_W + p
        i = j // EMB_DIM
        d = j % EMB_DIM
        vcp = pltpu.async_copy(tab_hbm.at[i, d], vocab_v, vsem)
        @pl.when(jnp.logical_or(p == 0, d == 0))
        def _():
            pltpu.sync_copy(idx_hbm.at[i], idx_v)
        vcp.wait()

        def do_chunk(c, carry2):
            b0 = c * OUT_CHUNK
            gc = p * N_OCH + c

            def run(buf, sem):
                @pl.when(gc >= 2)
                def _():
                    pltpu.make_async_copy(
                        out_hbm.at[j, pl.ds(b0, OUT_CHUNK)], buf, sem).wait()

                @plsc.parallel_loop(0, OUT_CHUNK, step=16, unroll=8)
                def _gather(off):
                    iv = idx_v[pl.ds(b0 + off, 16)]
                    buf[pl.ds(off, 16)] = plsc.load_gather(vocab_v, [iv])

                pltpu.async_copy(buf, out_hbm.at[j, pl.ds(b0, OUT_CHUNK)], sem)

            @pl.when(c % 2 == 0)
            def _():
                run(ob0, os0)

            @pl.when(c % 2 == 1)
            def _():
                run(ob1, os1)

            return carry2

        lax.fori_loop(0, N_OCH, do_chunk, 0)
        return carry

    lax.fori_loop(0, PER_W, do_pair, 0)
    last_j = wid * PER_W + PER_W - 1
    pltpu.make_async_copy(
        out_hbm.at[last_j, pl.ds(0, OUT_CHUNK)], ob0, os0).wait()
    pltpu.make_async_copy(
        out_hbm.at[last_j, pl.ds(0, OUT_CHUNK)], ob1, os1).wait()


_sc_col_gather = pl.kernel(
    _body,
    out_type=jax.ShapeDtypeStruct((N_PAIRS, BATCH), jnp.float32),
    mesh=plsc.VectorSubcoreMesh(
        core_axis_name="c", subcore_axis_name="s",
        num_cores=NC, num_subcores=NS),
    scratch_types=[
        pltpu.VMEM((VOCAB,), jnp.float32),
        pltpu.VMEM((BATCH,), jnp.int32),
        pltpu.VMEM((OUT_CHUNK,), jnp.float32),
        pltpu.VMEM((OUT_CHUNK,), jnp.float32),
        pltpu.SemaphoreType.DMA,
        pltpu.SemaphoreType.DMA,
        pltpu.SemaphoreType.DMA,
    ],
    compiler_params=pltpu.CompilerParams(
        use_tc_tiling_on_sc=True, needs_layout_passes=False),
)


@jax.jit
def kernel(categorical_data, tables):
    cat_t = categorical_data.astype(jnp.int32).T
    tab_t = lax.transpose(tables, (0, 2, 1))
    out_t = _sc_col_gather(tab_t, cat_t)
    return out_t.T
